# baseline (speedup 1.0000x reference)
import jax
import jax.numpy as jnp
from jax import lax
from jax.experimental import pallas as pl
from jax.experimental.pallas import tpu as pltpu
from jax.experimental.pallas import tpu_sc as plsc

_LG = 128
_B = 4000
_PAD = 32


def _tc_body(q_ref, ones_ref, m_ref, os_ref, oi_ref):
    i = pl.program_id(0)
    block = m_ref[...]
    b, d = block.shape
    nchunk = d // _LG
    pd = block[:, 0:_LG] * q_ref[0]
    pn = block[:, 0:_LG] * block[:, 0:_LG]
    for c in range(1, nchunk):
        col = block[:, c * _LG:(c + 1) * _LG]
        pd = pd + col * q_ref[c]
        pn = pn + col * col
    ones = ones_ref[...]
    cdims = (((1,), (1,)), ((), ()))
    dots = lax.dot_general(ones, pd, cdims,
                           preferred_element_type=jnp.float32)
    nrm = lax.dot_general(ones, pn, cdims,
                          preferred_element_type=jnp.float32)
    score = dots * jnp.abs(dots) / jnp.maximum(nrm, 1e-30)
    bmax = jnp.max(score)
    iot = lax.broadcasted_iota(jnp.int32, score.shape, 1)
    cand = jnp.where(score == bmax, iot, jnp.int32(2**31 - 1))
    bidx = jnp.min(cand)
    os_ref[0, 0, 0] = bmax
    oi_ref[0, 0, 0] = i * b + bidx


def _pick(va, ia, vb, ib):
    better = (va > vb) | ((va == vb) & (ia < ib))
    return jnp.where(better, va, vb), jnp.where(better, ia, ib)


def _sc_body(s_hbm, i_hbm, m_hbm, out_hbm, sv, iv, ts, ti, idxv, rowv, sem):
    c = lax.axis_index("c")
    s = lax.axis_index("s")
    wid = s * 2 + c

    pltpu.sync_copy(s_hbm, sv)
    pltpu.sync_copy(i_hbm, iv)
    ts[pl.ds(16, 16)] = jnp.full((16,), -jnp.inf, jnp.float32)
    ti[pl.ds(16, 16)] = jnp.zeros((16,), jnp.int32)
    v, idx = _pick(sv[pl.ds(0, 16)], iv[pl.ds(0, 16)],
                   sv[pl.ds(16, 16)], iv[pl.ds(16, 16)])
    for off in (8, 4, 2, 1):
        ts[pl.ds(0, 16)] = v
        ti[pl.ds(0, 16)] = idx
        v, idx = _pick(v, idx, ts[pl.ds(off, 16)], ti[pl.ds(off, 16)])
    idxv[...] = idx

    @pl.when(wid == 0)
    def _():
        cp = pltpu.make_async_copy(m_hbm.at[idxv], rowv, sem)
        cp.start()
        cp.wait()
        pltpu.sync_copy(rowv.at[0], out_hbm)


@jax.jit
def kernel(query, memory):
    k, d = memory.shape
    assert k % _B == 0 and d % _LG == 0
    grid = k // _B
    q2 = query.reshape(d // _LG, _LG)
    ones = jnp.ones((1, _LG), jnp.float32)
    scores, idxs = pl.pallas_call(
        _tc_body,
        grid=(grid,),
        in_specs=[
            pl.BlockSpec((d // _LG, _LG), lambda i: (0, 0)),
            pl.BlockSpec((1, _LG), lambda i: (0, 0)),
            pl.BlockSpec((_B, d), lambda i: (i, 0)),
        ],
        out_specs=[
            pl.BlockSpec((1, 1, 1), lambda i: (i, 0, 0), memory_space=pltpu.SMEM),
            pl.BlockSpec((1, 1, 1), lambda i: (i, 0, 0), memory_space=pltpu.SMEM),
        ],
        out_shape=[
            jax.ShapeDtypeStruct((grid, 1, 1), jnp.float32),
            jax.ShapeDtypeStruct((grid, 1, 1), jnp.int32),
        ],
    )(q2, ones, memory)

    s_pad = jnp.concatenate(
        [scores.reshape(grid), jnp.full((_PAD - grid,), -jnp.inf, jnp.float32)])
    i_pad = jnp.concatenate(
        [idxs.reshape(grid), jnp.zeros((_PAD - grid,), jnp.int32)])

    mesh = plsc.VectorSubcoreMesh(core_axis_name="c", subcore_axis_name="s")
    merge = pl.kernel(
        _sc_body,
        out_type=jax.ShapeDtypeStruct((d,), jnp.float32),
        mesh=mesh,
        scratch_types=[
            pltpu.VMEM((_PAD,), jnp.float32),
            pltpu.VMEM((_PAD,), jnp.int32),
            pltpu.VMEM((32,), jnp.float32),
            pltpu.VMEM((32,), jnp.int32),
            pltpu.VMEM((16,), jnp.int32),
            pltpu.VMEM((16, d), jnp.float32),
            pltpu.SemaphoreType.DMA,
        ],
    )
    return merge(s_pad, i_pad, memory)
